# baseline (speedup 1.0000x reference)
import functools
import jax
import jax.numpy as jnp
from jax import lax
from jax.experimental import pallas as pl
from jax.experimental.pallas import tpu as pltpu

TOKEN_BLOCK = 256
NBUF = 4
ROUTER_TOP_K = 2
ROUTER_CAPACITY_FACTOR = 1.0


def _router_block(x_ref, w_ref, disp_ref, gates_ref, eidx_ref,
                  buf_ref, sems, carry_ref, *, capacity, num_experts):
    T = x_ref.shape[0]
    E = num_experts
    i = pl.program_id(0)
    nsteps = pl.num_programs(0)

    @pl.when(i == 0)
    def _init():
        carry_ref[...] = jnp.zeros_like(carry_ref)

    logits = jnp.dot(x_ref[...], w_ref[...],
                     preferred_element_type=jnp.float32)

    iota_e = lax.broadcasted_iota(jnp.int32, (T, E), 1)
    m0 = jnp.max(logits, axis=1, keepdims=True)
    e0 = jnp.min(jnp.where(logits == m0, iota_e, E), axis=1, keepdims=True)
    mask0 = iota_e == e0
    l2 = jnp.where(mask0, -jnp.inf, logits)
    m1 = jnp.max(l2, axis=1, keepdims=True)
    e1 = jnp.min(jnp.where(l2 == m1, iota_e, E), axis=1, keepdims=True)
    mask1 = iota_e == e1

    t = jnp.exp(m1 - m0)
    denom = 1.0 + t
    row0 = i * T
    gates_ref[pl.ds(row0, T), :] = jnp.concatenate([1.0 / denom, t / denom],
                                                   axis=1)
    eidx_ref[pl.ds(row0, T), :] = jnp.concatenate([e0, e1], axis=1)

    ir = lax.broadcasted_iota(jnp.int32, (T, T), 0)
    ic = lax.broadcasted_iota(jnp.int32, (T, T), 1)
    tri = (ir >= ic).astype(jnp.float32)
    m0f = mask0.astype(jnp.float32)
    m1f = mask1.astype(jnp.float32)
    c0 = jnp.dot(tri, m0f, preferred_element_type=jnp.float32)
    c1 = jnp.dot(tri, m1f, preferred_element_type=jnp.float32)
    p0 = c0 + carry_ref[0:1, :]
    p1 = c1 + carry_ref[1:2, :]
    carry_ref[0:1, :] = carry_ref[0:1, :] + c0[T - 1:T, :]
    carry_ref[1:2, :] = carry_ref[1:2, :] + c1[T - 1:T, :]

    pos0 = jnp.sum(m0f * p0, axis=1, keepdims=True) - 1.0
    pos1 = jnp.sum(m1f * p1, axis=1, keepdims=True) - 1.0
    col0 = e0 * capacity + pos0.astype(jnp.int32)
    col1 = e1 * capacity + pos1.astype(jnp.int32)
    col0 = jnp.where(pos0 < capacity, col0, -1)
    col1 = jnp.where(pos1 < capacity, col1, -1)
    e_iota = lax.broadcasted_iota(jnp.int32, (T, E, capacity), 1)
    c_iota = lax.broadcasted_iota(jnp.int32, (T, E, capacity), 2)
    col3 = e_iota * capacity + c_iota
    d = (col3 == col0[:, :, None]) | (col3 == col1[:, :, None])
    df = d.astype(jnp.float32)

    slot = lax.rem(i, NBUF)

    @pl.when(i >= NBUF)
    def _drain_slot():
        pltpu.make_async_copy(
            buf_ref.at[slot],
            disp_ref.at[pl.ds((i - NBUF) * T, T)],
            sems.at[slot],
        ).wait()

    for b in range(NBUF):
        @pl.when(slot == b)
        def _store(b=b):
            buf_ref[b] = df

    pltpu.make_async_copy(
        buf_ref.at[slot],
        disp_ref.at[pl.ds(i * T, T)],
        sems.at[slot],
    ).start()

    @pl.when(i == nsteps - 1)
    def _drain_all():
        for b in range(NBUF):
            pltpu.make_async_copy(
                buf_ref.at[b],
                disp_ref.at[pl.ds(0, T)],
                sems.at[b],
            ).wait()


def kernel(x, W):
    b, s, d = x.shape
    e = W.shape[1]
    nt = b * s
    capacity = int(nt / e * ROUTER_CAPACITY_FACTOR)
    T = TOKEN_BLOCK
    xf = x.reshape(nt, d)
    disp, gates, eidx = pl.pallas_call(
        functools.partial(_router_block, capacity=capacity, num_experts=e),
        grid=(nt // T,),
        in_specs=[
            pl.BlockSpec((T, d), lambda i: (i, 0)),
            pl.BlockSpec((d, e), lambda i: (0, 0)),
        ],
        out_specs=[
            pl.BlockSpec(memory_space=pl.ANY),
            pl.BlockSpec((nt, ROUTER_TOP_K), lambda i: (0, 0)),
            pl.BlockSpec((nt, ROUTER_TOP_K), lambda i: (0, 0)),
        ],
        out_shape=[
            jax.ShapeDtypeStruct((nt, e, capacity), jnp.float32),
            jax.ShapeDtypeStruct((nt, ROUTER_TOP_K), jnp.float32),
            jax.ShapeDtypeStruct((nt, ROUTER_TOP_K), jnp.int32),
        ],
        scratch_shapes=[
            pltpu.VMEM((NBUF, T, e, capacity), jnp.float32),
            pltpu.SemaphoreType.DMA((NBUF,)),
            pltpu.VMEM((ROUTER_TOP_K, e), jnp.float32),
        ],
        compiler_params=pltpu.CompilerParams(
            dimension_semantics=("arbitrary",),
        ),
    )(xf, W)
    return (
        disp,
        gates.reshape(b, s, ROUTER_TOP_K),
        eidx.reshape(b, s, ROUTER_TOP_K),
    )


# speedup vs baseline: 1.2143x; 1.2143x over previous
import functools
import jax
import jax.numpy as jnp
from jax import lax
from jax.experimental import pallas as pl
from jax.experimental.pallas import tpu as pltpu

TOKEN_BLOCK = 1024
ROUTER_TOP_K = 2
ROUTER_CAPACITY_FACTOR = 1.0


def _router_block(x_ref, w_ref, disp_ref, gates_ref, eidx_ref, carry_ref,
                  *, capacity, num_experts):
    T = x_ref.shape[0]
    E = num_experts

    @pl.when(pl.program_id(0) == 0)
    def _init():
        carry_ref[...] = jnp.zeros_like(carry_ref)

    logits = jnp.dot(x_ref[...], w_ref[...],
                     preferred_element_type=jnp.float32)

    iota_e = lax.broadcasted_iota(jnp.int32, (T, E), 1)
    m0 = jnp.max(logits, axis=1, keepdims=True)
    e0 = jnp.min(jnp.where(logits == m0, iota_e, E), axis=1, keepdims=True)
    mask0 = iota_e == e0
    l2 = jnp.where(mask0, -jnp.inf, logits)
    m1 = jnp.max(l2, axis=1, keepdims=True)
    e1 = jnp.min(jnp.where(l2 == m1, iota_e, E), axis=1, keepdims=True)
    mask1 = iota_e == e1

    t = jnp.exp(m1 - m0)
    denom = 1.0 + t
    gates_ref[...] = jnp.concatenate([1.0 / denom, t / denom], axis=1)
    eidx_ref[...] = jnp.concatenate([e0, e1], axis=1)

    ir = lax.broadcasted_iota(jnp.int32, (T, T), 0)
    ic = lax.broadcasted_iota(jnp.int32, (T, T), 1)
    tri = (ir >= ic).astype(jnp.float32)
    m0f = mask0.astype(jnp.float32)
    m1f = mask1.astype(jnp.float32)
    c0 = jnp.dot(tri, m0f, preferred_element_type=jnp.float32)
    c1 = jnp.dot(tri, m1f, preferred_element_type=jnp.float32)
    p0 = c0 + carry_ref[0:1, :]
    p1 = c1 + carry_ref[1:2, :]
    carry_ref[0:1, :] = carry_ref[0:1, :] + c0[T - 1:T, :]
    carry_ref[1:2, :] = carry_ref[1:2, :] + c1[T - 1:T, :]

    pos0 = jnp.sum(m0f * p0, axis=1, keepdims=True) - 1.0
    pos1 = jnp.sum(m1f * p1, axis=1, keepdims=True) - 1.0
    col0 = e0 * capacity + pos0.astype(jnp.int32)
    col1 = e1 * capacity + pos1.astype(jnp.int32)
    col0 = jnp.where(pos0 < capacity, col0, -1)
    col1 = jnp.where(pos1 < capacity, col1, -1)
    e_iota = lax.broadcasted_iota(jnp.int32, (T, E, capacity), 1)
    c_iota = lax.broadcasted_iota(jnp.int32, (T, E, capacity), 2)
    col3 = e_iota * capacity + c_iota
    d = (col3 == col0[:, :, None]) | (col3 == col1[:, :, None])
    disp_ref[...] = d.astype(jnp.float32)


def kernel(x, W):
    b, s, d = x.shape
    e = W.shape[1]
    nt = b * s
    capacity = int(nt / e * ROUTER_CAPACITY_FACTOR)
    T = TOKEN_BLOCK
    xf = x.reshape(nt, d)
    disp, gates, eidx = pl.pallas_call(
        functools.partial(_router_block, capacity=capacity, num_experts=e),
        grid=(nt // T,),
        in_specs=[
            pl.BlockSpec((T, d), lambda i: (i, 0)),
            pl.BlockSpec((d, e), lambda i: (0, 0)),
        ],
        out_specs=[
            pl.BlockSpec((T, e, capacity), lambda i: (i, 0, 0)),
            pl.BlockSpec((T, ROUTER_TOP_K), lambda i: (i, 0)),
            pl.BlockSpec((T, ROUTER_TOP_K), lambda i: (i, 0)),
        ],
        out_shape=[
            jax.ShapeDtypeStruct((nt, e, capacity), jnp.float32),
            jax.ShapeDtypeStruct((nt, ROUTER_TOP_K), jnp.float32),
            jax.ShapeDtypeStruct((nt, ROUTER_TOP_K), jnp.int32),
        ],
        scratch_shapes=[pltpu.VMEM((ROUTER_TOP_K, e), jnp.float32)],
        compiler_params=pltpu.CompilerParams(
            dimension_semantics=("arbitrary",),
        ),
    )(xf, W)
    return (
        disp,
        gates.reshape(b, s, ROUTER_TOP_K),
        eidx.reshape(b, s, ROUTER_TOP_K),
    )
